# baseline (speedup 1.0000x reference)
import functools

import jax
import jax.numpy as jnp
from jax import lax
from jax.experimental import pallas as pl
from jax.experimental.pallas import tpu as pltpu
from jax.experimental.pallas import tpu_sc as plsc

_CHUNK = 128
_NSUB = 16
_LANES = 16
_NBUF = 4


@functools.lru_cache(maxsize=None)
def _make_sc_spmm(n_nodes, d_half, n_chunks_per_tile):
  assert n_chunks_per_tile % _NBUF == 0
  mesh = plsc.VectorSubcoreMesh(core_axis_name="c", subcore_axis_name="s")
  rows_per_tile = n_nodes // _NSUB
  last = n_chunks_per_tile - 1

  @functools.partial(
      pl.kernel,
      mesh=mesh,
      out_type=jax.ShapeDtypeStruct((n_nodes, 2, d_half), jnp.float32),
      compiler_params=pltpu.CompilerParams(
          use_tc_tiling_on_sc=False, needs_layout_passes=False),
      scratch_types=[
          pltpu.VMEM_SHARED((n_nodes, d_half), jnp.float32),
          pltpu.VMEM_SHARED((n_nodes, d_half), jnp.float32),
          [pltpu.VMEM((3, _CHUNK), jnp.int32) for _ in range(_NBUF)],
          [pltpu.VMEM((_CHUNK, d_half), jnp.float32) for _ in range(_NBUF)],
          [pltpu.SemaphoreType.DMA for _ in range(_NBUF)],
          [pltpu.SemaphoreType.DMA for _ in range(_NBUF)],
          [pltpu.SemaphoreType.DMA for _ in range(_NBUF)],
      ],
  )
  def spmm(xv_hbm, pk_hbm, zero_hbm, out_hbm,
           acc, x_sp, idx, msg, isem, gsem, ssem):
    c = lax.axis_index("c")
    s = lax.axis_index("s")

    r0 = s * rows_per_tile
    pltpu.sync_copy(xv_hbm.at[pl.ds(r0, rows_per_tile), c],
                    x_sp.at[pl.ds(r0, rows_per_tile)])
    pltpu.sync_copy(zero_hbm.at[pl.ds(r0, rows_per_tile)],
                    acc.at[pl.ds(r0, rows_per_tile)])
    plsc.subcore_barrier()

    def load_idx(g, b):
      pltpu.async_copy(pk_hbm.at[s, g], idx[b], isem[b])

    def start_gather(g, b):
      pltpu.async_copy(x_sp.at[idx[b].at[1]], msg[b], gsem[b])

    def scale(b):
      mref = msg[b]
      iref = idx[b]

      def ubody(u, _):
        a = plsc.bitcast(iref[2, pl.ds(u * _LANES, _LANES)], jnp.float32)
        dnums = lax.GatherDimensionNumbers(
            offset_dims=(), collapsed_slice_dims=(0,), start_index_map=(0,))
        for kk in range(_LANES):
          av = lax.gather(a, jnp.full((_LANES, 1), kk, jnp.int32), dnums,
                          slice_sizes=(1,),
                          mode=lax.GatherScatterMode.PROMISE_IN_BOUNDS)
          k = u * _LANES + kk
          for j in range(d_half // _LANES):
            sl = pl.ds(j * _LANES, _LANES)
            mref[k, sl] = mref[k, sl] * av
        return 0

      lax.fori_loop(0, _CHUNK // _LANES, ubody, 0)

    load_idx(0, 0)
    load_idx(1, 1)
    pltpu.make_async_copy(pk_hbm.at[s, 0], idx[0], isem[0]).wait()
    start_gather(0, 0)

    def ring_body(i, _):
      for bb in range(_NBUF):
        g = _NBUF * i + bb
        b = bb
        b1 = (bb + 1) % _NBUF
        b2 = (bb + 2) % _NBUF

        @pl.when(g >= 2)
        def _():
          pltpu.make_async_copy(msg[b2], acc.at[idx[b2].at[0]],
                                ssem[b2]).wait()

        @pl.when(g + 2 <= last)
        def _():
          load_idx(g + 2, b2)

        @pl.when(g + 1 <= last)
        def _():
          pltpu.make_async_copy(pk_hbm.at[s, g + 1], idx[b1], isem[b1]).wait()
          start_gather(g + 1, b1)

        pltpu.make_async_copy(x_sp.at[idx[b].at[1]], msg[b], gsem[b]).wait()
        scale(b)
        pltpu.async_copy(msg[b], acc.at[idx[b].at[0]], ssem[b], add=True)
      return 0

    lax.fori_loop(0, n_chunks_per_tile // _NBUF, ring_body, 0)
    for g in (last - 1, last):
      b = g % _NBUF
      pltpu.make_async_copy(msg[b], acc.at[idx[b].at[0]], ssem[b]).wait()

    plsc.subcore_barrier()
    pltpu.sync_copy(acc.at[pl.ds(r0, rows_per_tile)],
                    out_hbm.at[pl.ds(r0, rows_per_tile), c])

  return spmm


def kernel(x, edge_index, edge_attr):
  n, d = x.shape
  e = edge_attr.shape[0]
  row = edge_index[0].astype(jnp.int32)
  col = edge_index[1].astype(jnp.int32)
  attr_bits = jax.lax.bitcast_convert_type(
      edge_attr.astype(jnp.float32), jnp.int32)

  n_chunks_per_tile = -(-e // (_NSUB * _CHUNK))
  n_chunks_per_tile += (-n_chunks_per_tile) % _NBUF
  e_pad = n_chunks_per_tile * _NSUB * _CHUNK
  pad = e_pad - e
  if pad:
    row = jnp.concatenate([row, jnp.zeros((pad,), jnp.int32)])
    col = jnp.concatenate([col, jnp.zeros((pad,), jnp.int32)])
    attr_bits = jnp.concatenate([attr_bits, jnp.zeros((pad,), jnp.int32)])

  pk = jnp.stack([row.reshape(_NSUB, n_chunks_per_tile, _CHUNK),
                  col.reshape(_NSUB, n_chunks_per_tile, _CHUNK),
                  attr_bits.reshape(_NSUB, n_chunks_per_tile, _CHUNK)],
                 axis=2)

  xv = x.reshape(n, 2, d // 2)
  zero = jnp.zeros((n, d // 2), jnp.float32)
  out = _make_sc_spmm(n, d // 2, n_chunks_per_tile)(xv, pk, zero)
  return out.reshape(n, d)
